# baseline (speedup 1.0000x reference)
import functools

import jax
import jax.numpy as jnp
from jax import lax
from jax.experimental import pallas as pl
from jax.experimental.pallas import tpu as pltpu
from jax.experimental.pallas import tpu_sc as plsc

NUM_ENTITIES = 1000000
EMBEDDING_DIM = 64
BATCH = 16384

_NUM_CORES = 2
_NUM_SUBCORES = 16
_NUM_WORKERS = _NUM_CORES * _NUM_SUBCORES
_B_PER_W = BATCH // _NUM_WORKERS
_PAIR_DIM = 2 * EMBEDDING_DIM

_mesh = plsc.VectorSubcoreMesh(core_axis_name="c", subcore_axis_name="s")


@functools.partial(
    pl.kernel,
    mesh=_mesh,
    out_type=jax.ShapeDtypeStruct((BATCH, EMBEDDING_DIM), jnp.float32),
    scratch_types=[
        pltpu.VMEM((_B_PER_W,), jnp.int32),
        pltpu.VMEM((_B_PER_W,), jnp.int32),
        pltpu.VMEM((_B_PER_W, _PAIR_DIM), jnp.float32),
        pltpu.VMEM((_B_PER_W, EMBEDDING_DIM), jnp.float32),
        pltpu.SemaphoreType.DMA,
    ],
    compiler_params=pltpu.CompilerParams(use_tc_tiling_on_sc=False),
)
def _gather_kernel(idx_hbm, tab2_hbm, out_hbm, idx_v, idx2_v, pairs_v, rows_v, sem):
    wid = lax.axis_index("s") * _NUM_CORES + lax.axis_index("c")
    base = wid * _B_PER_W
    pltpu.sync_copy(idx_hbm.at[pl.ds(base, _B_PER_W)], idx_v)

    def make_pair_idx(c, carry):
        v = idx_v[pl.ds(c * 16, 16)]
        idx2_v[pl.ds(c * 16, 16)] = lax.shift_right_logical(v, 1)
        return carry

    lax.fori_loop(0, _B_PER_W // 16, make_pair_idx, 0)

    pltpu.async_copy(tab2_hbm.at[idx2_v], pairs_v, sem).wait()

    def select_half(c, carry):
        off = (idx_v[pl.ds(c * 16, 16)] & 1) * EMBEDDING_DIM
        for l in range(16):
            j = c * 16 + l
            s = off[l]
            for k in range(4):
                rows_v[j, pl.ds(k * 16, 16)] = pairs_v[j, pl.ds(s + k * 16, 16)]
        return carry

    lax.fori_loop(0, _B_PER_W // 16, select_half, 0)

    pltpu.sync_copy(rows_v, out_hbm.at[pl.ds(base, _B_PER_W)])


@jax.jit
def kernel(entities, entity_embeddings):
    table2 = entity_embeddings.reshape(NUM_ENTITIES // 2, _PAIR_DIM)
    return _gather_kernel(entities, table2)


# speedup vs baseline: 1.7667x; 1.7667x over previous
import functools

import jax
import jax.numpy as jnp
from jax import lax
from jax.experimental import pallas as pl
from jax.experimental.pallas import tpu as pltpu
from jax.experimental.pallas import tpu_sc as plsc

NUM_ENTITIES = 1000000
EMBEDDING_DIM = 64
BATCH = 16384

_NUM_CORES = 2
_NUM_SUBCORES = 16
_NUM_WORKERS = _NUM_CORES * _NUM_SUBCORES
_B_PER_W = BATCH // _NUM_WORKERS

_mesh = plsc.VectorSubcoreMesh(core_axis_name="c", subcore_axis_name="s")


@functools.partial(
    pl.kernel,
    mesh=_mesh,
    out_type=jax.ShapeDtypeStruct((BATCH, EMBEDDING_DIM), jnp.float32),
    scratch_types=[
        pltpu.VMEM((_B_PER_W,), jnp.int32),
        pltpu.VMEM((_B_PER_W, EMBEDDING_DIM), jnp.float32),
        pltpu.SemaphoreType.DMA,
        pltpu.SemaphoreType.DMA,
    ],
    compiler_params=pltpu.CompilerParams(use_tc_tiling_on_sc=True),
)
def _gather_kernel(idx_hbm, table_hbm, out_hbm, idx_v, rows_v, sem, rsem):
    wid = lax.axis_index("s") * _NUM_CORES + lax.axis_index("c")
    base = wid * _B_PER_W
    pltpu.sync_copy(idx_hbm.at[pl.ds(base, _B_PER_W)], idx_v)

    def fire(c, carry):
        vec = idx_v[pl.ds(c * 16, 16)]
        for l in range(16):
            i = vec[l]
            pltpu.async_copy(
                table_hbm.at[pl.ds(i, 1), :],
                rows_v.at[pl.ds(c * 16 + l, 1), :],
                rsem,
            )
        return carry

    lax.fori_loop(0, _B_PER_W // 16, fire, 0)

    pltpu.make_async_copy(
        table_hbm.at[pl.ds(0, _B_PER_W), :], rows_v, rsem
    ).wait()

    pltpu.sync_copy(rows_v, out_hbm.at[pl.ds(base, _B_PER_W)])


@jax.jit
def kernel(entities, entity_embeddings):
    return _gather_kernel(entities, entity_embeddings)
